# baseline (speedup 1.0000x reference)
import functools
import math

import jax
import jax.numpy as jnp
from jax import lax
from jax.experimental import pallas as pl
from jax.experimental.pallas import tpu as pltpu
from jax.experimental.pallas import tpu_sc as plsc

EMB_DIM = 64
SCALE = math.sqrt(EMB_DIM)

NC = 2
NS = 16
NW = NC * NS
LANES = 16

BW = 128
GRP = 8
NBUF = 4
PITCH = BW + 1


def _make_kernel(s_len, n_win):
    mesh = plsc.VectorSubcoreMesh(core_axis_name="c", subcore_axis_name="s")
    n_grp = EMB_DIM // GRP

    @functools.partial(
        pl.kernel,
        out_type=jax.ShapeDtypeStruct((s_len, n_grp, n_win, GRP, BW), jnp.float32),
        mesh=mesh,
        scratch_types=[
            pltpu.VMEM((s_len, BW), jnp.int32),
            [pltpu.VMEM((BW, 2 * EMB_DIM), jnp.float32) for _ in range(NBUF)],
            [pltpu.VMEM((GRP, GRP, PITCH), jnp.float32) for _ in range(NBUF)],
            [pltpu.SemaphoreType.DMA for _ in range(NBUF)],
            [pltpu.SemaphoreType.DMA for _ in range(NBUF)],
        ],
        compiler_params=pltpu.CompilerParams(
            use_tc_tiling_on_sc=False, needs_layout_passes=False
        ),
    )
    def gather_tr(table_hbm, tok_hbm, out_hbm, idx_v, stg, obuf, sg, sw):
        w = lax.axis_index("s") * NC + lax.axis_index("c")
        pltpu.sync_copy(tok_hbm.at[:, pl.ds(w * BW, BW)], idx_v)

        d_iota = lax.iota(jnp.int32, LANES)

        def fire_gather(s, b):
            pltpu.async_copy(table_hbm.at[idx_v.at[s]], stg[b], sg[b])

        def wait_gather(b):
            pltpu.make_async_copy(table_hbm.at[idx_v.at[0]], stg[b], sg[b]).wait()

        def fire_write(s, b):
            pltpu.async_copy(
                obuf[b].at[:, :, pl.ds(0, BW)], out_hbm.at[s, :, w], sw[b]
            )

        def wait_write(b):
            pltpu.make_async_copy(
                obuf[b].at[:, :, pl.ds(0, BW)], out_hbm.at[0, :, w], sw[b]
            ).wait()

        e_bases = [d_iota + (ev * LANES) for ev in range(EMB_DIM // LANES)]
        g_bases = [eb // GRP for eb in e_bases]
        r_bases = [eb % GRP for eb in e_bases]

        def transpose(b):
            @plsc.parallel_loop(0, BW, unroll=4)
            def d_body(d):
                d_splat = jnp.full((LANES,), 0, jnp.int32) + d
                for ev in range(EMB_DIM // LANES):
                    vals = stg[b][d, pl.ds(ev * LANES, LANES)] * SCALE
                    plsc.store_scatter(
                        obuf[b], [g_bases[ev], r_bases[ev], d_splat], vals
                    )

        for b in range(NBUF):
            fire_gather(b, b)
        for b in range(NBUF):
            wait_gather(b)
            transpose(b)
            fire_write(b, b)
            fire_gather(b + NBUF, b)

        def main_body(jj, _):
            for b in range(NBUF):
                s = jj * NBUF + b
                wait_gather(b)
                wait_write(b)
                transpose(b)
                fire_write(s, b)
                fire_gather(s + NBUF, b)
            return 0

        lax.fori_loop(1, s_len // NBUF - 1, main_body, 0)

        for b in range(NBUF):
            s = s_len - NBUF + b
            wait_gather(b)
            wait_write(b)
            transpose(b)
            fire_write(s, b)
        for b in range(NBUF):
            wait_write(b)

    return gather_tr


@jax.jit
def kernel(tokens, table):
    b, s = tokens.shape
    assert b % BW == 0 and (b // BW) == NW and s % NBUF == 0
    tok_t = tokens.T.astype(jnp.int32)
    tab_pad = jnp.pad(table, ((0, 0), (0, EMB_DIM)))
    out5 = _make_kernel(s, b // BW)(tab_pad, tok_t)
    return out5.transpose(2, 4, 0, 1, 3).reshape(b, s, EMB_DIM)


# speedup vs baseline: 1.0088x; 1.0088x over previous
import functools
import math

import jax
import jax.numpy as jnp
from jax import lax
from jax.experimental import pallas as pl
from jax.experimental.pallas import tpu as pltpu
from jax.experimental.pallas import tpu_sc as plsc

EMB_DIM = 64
SCALE = math.sqrt(EMB_DIM)

NC = 2
NS = 16
NW = NC * NS
LANES = 16

BW = 128
GRP = 8
NBUF = 4
PITCH = BW + 1


def _make_kernel(s_len, n_win):
    mesh = plsc.VectorSubcoreMesh(core_axis_name="c", subcore_axis_name="s")
    n_grp = EMB_DIM // GRP

    @functools.partial(
        pl.kernel,
        out_type=jax.ShapeDtypeStruct((s_len, n_grp, n_win, GRP, BW), jnp.float32),
        mesh=mesh,
        scratch_types=[
            pltpu.VMEM((s_len, BW), jnp.int32),
            [pltpu.VMEM((BW, EMB_DIM), jnp.float32) for _ in range(NBUF)],
            [pltpu.VMEM((GRP, GRP, PITCH), jnp.float32) for _ in range(NBUF)],
            [pltpu.SemaphoreType.DMA for _ in range(NBUF)],
            [pltpu.SemaphoreType.DMA for _ in range(NBUF)],
        ],
        compiler_params=pltpu.CompilerParams(
            use_tc_tiling_on_sc=False, needs_layout_passes=False
        ),
    )
    def gather_tr(table_hbm, tok_hbm, out_hbm, idx_v, stg, obuf, sg, sw):
        w = lax.axis_index("s") * NC + lax.axis_index("c")
        pltpu.sync_copy(tok_hbm.at[:, pl.ds(w * BW, BW)], idx_v)

        d_iota = lax.iota(jnp.int32, LANES)

        def fire_gather(s, b):
            pltpu.async_copy(table_hbm.at[idx_v.at[s]], stg[b], sg[b])

        def wait_gather(b):
            pltpu.make_async_copy(table_hbm.at[idx_v.at[0]], stg[b], sg[b]).wait()

        def fire_write(s, b):
            pltpu.async_copy(
                obuf[b].at[:, :, pl.ds(0, BW)], out_hbm.at[s, :, w], sw[b]
            )

        def wait_write(b):
            pltpu.make_async_copy(
                obuf[b].at[:, :, pl.ds(0, BW)], out_hbm.at[0, :, w], sw[b]
            ).wait()

        e_bases = [d_iota + (ev * LANES) for ev in range(EMB_DIM // LANES)]
        g_bases = [eb // GRP for eb in e_bases]
        r_bases = [eb % GRP for eb in e_bases]

        def transpose(b):
            @plsc.parallel_loop(0, BW, unroll=8)
            def d_body(d):
                d_splat = jnp.full((LANES,), 0, jnp.int32) + d
                for ev in range(EMB_DIM // LANES):
                    vals = stg[b][d, pl.ds(ev * LANES, LANES)] * SCALE
                    plsc.store_scatter(
                        obuf[b], [g_bases[ev], r_bases[ev], d_splat], vals
                    )

        for b in range(NBUF):
            fire_gather(b, b)
        for b in range(NBUF):
            wait_gather(b)
            transpose(b)
            fire_write(b, b)
            fire_gather(b + NBUF, b)

        def main_body(jj, _):
            for b in range(NBUF):
                s = jj * NBUF + b
                wait_gather(b)
                wait_write(b)
                transpose(b)
                fire_write(s, b)
                fire_gather(s + NBUF, b)
            return 0

        lax.fori_loop(1, s_len // NBUF - 1, main_body, 0)

        for b in range(NBUF):
            s = s_len - NBUF + b
            wait_gather(b)
            wait_write(b)
            transpose(b)
            fire_write(s, b)
        for b in range(NBUF):
            wait_write(b)

    return gather_tr


@jax.jit
def kernel(tokens, table):
    b, s = tokens.shape
    assert b % BW == 0 and (b // BW) == NW and s % NBUF == 0
    tok_t = tokens.T.astype(jnp.int32)
    out5 = _make_kernel(s, b // BW)(table, tok_t)
    return out5.transpose(2, 4, 0, 1, 3).reshape(b, s, EMB_DIM)


# speedup vs baseline: 1.0159x; 1.0070x over previous
import functools
import math

import jax
import jax.numpy as jnp
from jax import lax
from jax.experimental import pallas as pl
from jax.experimental.pallas import tpu as pltpu
from jax.experimental.pallas import tpu_sc as plsc

EMB_DIM = 64
SCALE = math.sqrt(EMB_DIM)

NC = 2
NS = 16
NW = NC * NS
LANES = 16

BW = 128
GRP = 8
NBUF = 4
PITCH = BW + 1


def _make_kernel(s_len, n_win):
    mesh = plsc.VectorSubcoreMesh(core_axis_name="c", subcore_axis_name="s")
    n_grp = EMB_DIM // GRP

    @functools.partial(
        pl.kernel,
        out_type=jax.ShapeDtypeStruct((s_len, n_grp, n_win, GRP, BW), jnp.float32),
        mesh=mesh,
        scratch_types=[
            pltpu.VMEM((s_len, BW), jnp.int32),
            [pltpu.VMEM((BW, EMB_DIM), jnp.float32) for _ in range(NBUF)],
            [pltpu.VMEM((GRP, GRP, PITCH), jnp.float32) for _ in range(NBUF)],
            [pltpu.SemaphoreType.DMA for _ in range(NBUF)],
            [pltpu.SemaphoreType.DMA for _ in range(NBUF)],
        ],
        compiler_params=pltpu.CompilerParams(
            use_tc_tiling_on_sc=False, needs_layout_passes=False
        ),
    )
    def gather_tr(table_hbm, tok_hbm, out_hbm, idx_v, stg, obuf, sg, sw):
        w = lax.axis_index("s") * NC + lax.axis_index("c")
        pltpu.sync_copy(tok_hbm.at[:, pl.ds(w * BW, BW)], idx_v)

        d_iota = lax.iota(jnp.int32, LANES)

        def fire_gather(s, b):
            pltpu.async_copy(table_hbm.at[idx_v.at[s]], stg[b], sg[b])

        def wait_gather(b):
            pltpu.make_async_copy(table_hbm.at[idx_v.at[0]], stg[b], sg[b]).wait()

        def fire_write(s, b):
            pltpu.async_copy(
                obuf[b].at[:, :, pl.ds(0, BW)], out_hbm.at[s, :, w], sw[b]
            )

        def wait_write(b):
            pltpu.make_async_copy(
                obuf[b].at[:, :, pl.ds(0, BW)], out_hbm.at[0, :, w], sw[b]
            ).wait()

        e_bases = [d_iota + (ev * LANES) for ev in range(EMB_DIM // LANES)]
        g_bases = [eb // GRP for eb in e_bases]
        r_bases = [eb % GRP for eb in e_bases]

        def transpose(b):
            @plsc.parallel_loop(0, BW, unroll=4)
            def d_body(d):
                d_splat = jnp.full((LANES,), 0, jnp.int32) + d
                for ev in range(EMB_DIM // LANES):
                    vals = stg[b][d, pl.ds(ev * LANES, LANES)] * SCALE
                    plsc.store_scatter(
                        obuf[b], [g_bases[ev], r_bases[ev], d_splat], vals
                    )

        for b in range(NBUF):
            fire_gather(b, b)
        for b in range(NBUF):
            wait_gather(b)
            transpose(b)
            fire_write(b, b)
            fire_gather(b + NBUF, b)

        def main_body(jj, _):
            for b in range(NBUF):
                s = jj * NBUF + b
                wait_gather(b)
                wait_write(b)
                transpose(b)
                fire_write(s, b)
                fire_gather(s + NBUF, b)
            return 0

        lax.fori_loop(1, s_len // NBUF - 1, main_body, 0)

        for b in range(NBUF):
            s = s_len - NBUF + b
            wait_gather(b)
            wait_write(b)
            transpose(b)
            fire_write(s, b)
        for b in range(NBUF):
            wait_write(b)

    return gather_tr


@jax.jit
def kernel(tokens, table):
    b, s = tokens.shape
    assert b % BW == 0 and (b // BW) == NW and s % NBUF == 0
    tok_t = tokens.T.astype(jnp.int32)
    out5 = _make_kernel(s, b // BW)(table, tok_t)
    return out5.transpose(2, 4, 0, 1, 3).reshape(b, s, EMB_DIM)
